# Initial kernel scaffold; baseline (speedup 1.0000x reference)
import jax
import jax.numpy as jnp
from jax.experimental import pallas as pl


def kernel(x, edge_index, W1_0, b1_0, g_0, beta_0, W2_0, b2_0, W1_1, b1_1, g_1, beta_1, W2_1, b2_1, W1_2, b1_2, g_2, beta_2, W2_2, b2_2):
    raise NotImplementedError("write your pallas kernel here")


# speedup vs baseline: 5.0773x; 5.0773x over previous
import functools

import jax
import jax.numpy as jnp
from jax import lax
from jax.experimental import pallas as pl
from jax.experimental.pallas import tpu as pltpu
from jax.experimental.pallas import tpu_sc as plsc

_N = 10000
_E = 320000
_D = 128
_H = 2 * _D
_EPS = 1e-7

_NC = 2
_NS = 16
_NW = _NC * _NS
_EPW = _E // _NW
_C = 128
_NCH = (_EPW + _C - 1) // _C
_EPWP = _NCH * _C
_NPAD = 10240
_TRASH = _N
_RPS = _NPAD // _NS

_sc_mesh = plsc.VectorSubcoreMesh(core_axis_name="c", subcore_axis_name="s")


def _zero_rows(rows_v, nrows):
    def zrow(i, carry):
        def zlane(j, carry2):
            rows_v[i, pl.ds(j * 16, 16)] = jnp.zeros((16,), jnp.float32)
            return carry2
        return lax.fori_loop(0, _D // 16, zlane, carry)
    lax.fori_loop(0, nrows, zrow, 0)


def _fill_ones(rows_v, nrows):
    def frow(i, carry):
        def flane(j, carry2):
            rows_v[i, pl.ds(j * 16, 16)] = jnp.ones((16,), jnp.float32)
            return carry2
        return lax.fori_loop(0, _D // 16, flane, carry)
    lax.fori_loop(0, nrows, frow, 0)


@functools.partial(
    pl.kernel,
    mesh=_sc_mesh,
    out_type=jax.ShapeDtypeStruct((_NC, _NPAD, _D), jnp.float32),
    scratch_types=[
        pltpu.VMEM((_NCH, _C), jnp.int32),
        pltpu.VMEM((_NCH, _C), jnp.int32),
        pltpu.VMEM((_C, _D), jnp.float32),
        pltpu.VMEM_SHARED((_NPAD, _D), jnp.float32),
        pltpu.SemaphoreType.DMA,
    ],
)
def _sc_seg_sum(y_hbm, src_hbm, dst_hbm, out_hbm, src_v, dst_v, rows_v, acc, sem):
    c = lax.axis_index("c")
    s = lax.axis_index("s")
    wid = c * _NS + s
    pltpu.sync_copy(src_hbm.at[wid], src_v)
    pltpu.sync_copy(dst_hbm.at[wid], dst_v)

    _zero_rows(rows_v, _C)
    for k in range(_RPS // _C):
        pltpu.sync_copy(rows_v, acc.at[pl.ds(s * _RPS + k * _C, _C)])
    plsc.subcore_barrier()

    def body(j, carry):
        pltpu.async_copy(y_hbm.at[src_v.at[j]], rows_v, sem).wait()
        pltpu.sync_copy(rows_v, acc.at[dst_v.at[j]], add=True)
        return carry
    lax.fori_loop(0, _NCH, body, 0)
    plsc.subcore_barrier()

    pltpu.sync_copy(
        acc.at[pl.ds(s * _RPS, _RPS)],
        out_hbm.at[c, pl.ds(s * _RPS, _RPS), :],
    )


@functools.partial(
    pl.kernel,
    mesh=_sc_mesh,
    out_type=jax.ShapeDtypeStruct((_NC, _NPAD, _D), jnp.float32),
    scratch_types=[
        pltpu.VMEM((_NCH, _C), jnp.int32),
        pltpu.VMEM((_C, _D), jnp.float32),
        pltpu.VMEM_SHARED((_NPAD, _D), jnp.float32),
    ],
)
def _sc_degree(dst_hbm, out_hbm, dst_v, rows_v, acc):
    c = lax.axis_index("c")
    s = lax.axis_index("s")
    wid = c * _NS + s
    pltpu.sync_copy(dst_hbm.at[wid], dst_v)

    _zero_rows(rows_v, _C)
    for k in range(_RPS // _C):
        pltpu.sync_copy(rows_v, acc.at[pl.ds(s * _RPS + k * _C, _C)])
    plsc.subcore_barrier()

    _fill_ones(rows_v, _C)

    def body(j, carry):
        pltpu.sync_copy(rows_v, acc.at[dst_v.at[j]], add=True)
        return carry
    lax.fori_loop(0, _NCH, body, 0)
    plsc.subcore_barrier()

    pltpu.sync_copy(
        acc.at[pl.ds(s * _RPS, _RPS)],
        out_hbm.at[c, pl.ds(s * _RPS, _RPS), :],
    )


_R = 1000


def _tc_prep(x):
    def body(x_ref, y_ref):
        y_ref[...] = jnp.maximum(x_ref[...], 0.0)
    return pl.pallas_call(
        body,
        grid=(_N // _R,),
        in_specs=[pl.BlockSpec((_R, _D), lambda i: (i, 0))],
        out_specs=pl.BlockSpec((_R, _D), lambda i: (i, 0)),
        out_shape=jax.ShapeDtypeStruct((_N, _D), jnp.float32),
    )(x)


def _mlp(out, W1_ref, b1_ref, g_ref, be_ref, W2_ref, b2_ref):
    h = jnp.dot(out, W1_ref[...], preferred_element_type=jnp.float32) + b1_ref[...]
    mu = jnp.mean(h, axis=1, keepdims=True)
    var = jnp.mean((h - mu) ** 2, axis=1, keepdims=True)
    hn = (h - mu) * lax.rsqrt(var + 1e-5) * g_ref[...] + be_ref[...]
    hr = jnp.maximum(hn, 0.0)
    return jnp.dot(hr, W2_ref[...], preferred_element_type=jnp.float32) + b2_ref[...]


def _tc_layer(sp, cp, h_prev, W1, b1, g, be, W2, b2, final):
    def body(s0_ref, s1_ref, c0_ref, c1_ref, hp_ref, W1_ref, b1_ref, g_ref,
             be_ref, W2_ref, b2_ref, *outs):
        sblk = s0_ref[0] + s1_ref[0]
        cnt = c0_ref[0] + c1_ref[0]
        inv = 1.0 / jnp.maximum(cnt, 1.0)
        agg = (sblk + _EPS * cnt) * inv
        out = agg + hp_ref[...]
        o = _mlp(out, W1_ref, b1_ref, g_ref, be_ref, W2_ref, b2_ref)
        if final:
            outs[0][...] = o
        else:
            hh = jnp.where(o > 0.0, o, jnp.exp(o) - 1.0)
            outs[0][...] = hh
            outs[1][...] = jnp.maximum(hh, 0.0)

    n_out = 1 if final else 2
    full = lambda r, c: pl.BlockSpec((r, c), lambda i: (0, 0))
    return pl.pallas_call(
        body,
        grid=(_N // _R,),
        in_specs=[
            pl.BlockSpec((1, _R, _D), lambda i: (0, i, 0)),
            pl.BlockSpec((1, _R, _D), lambda i: (1, i, 0)),
            pl.BlockSpec((1, _R, _D), lambda i: (0, i, 0)),
            pl.BlockSpec((1, _R, _D), lambda i: (1, i, 0)),
            pl.BlockSpec((_R, _D), lambda i: (i, 0)),
            full(_D, _H), full(1, _H), full(1, _H), full(1, _H),
            full(_H, _D), full(1, _D),
        ],
        out_specs=[pl.BlockSpec((_R, _D), lambda i: (i, 0))] * n_out,
        out_shape=[jax.ShapeDtypeStruct((_N, _D), jnp.float32)] * n_out,
    )(sp, sp, cp, cp, h_prev, W1, b1.reshape(1, -1), g.reshape(1, -1),
      be.reshape(1, -1), W2, b2.reshape(1, -1))


def kernel(x, edge_index,
           W1_0, b1_0, g_0, beta_0, W2_0, b2_0,
           W1_1, b1_1, g_1, beta_1, W2_1, b2_1,
           W1_2, b1_2, g_2, beta_2, W2_2, b2_2):
    src = edge_index[0].reshape(_NW, _EPW)
    dst = edge_index[1].reshape(_NW, _EPW)
    pad = _EPWP - _EPW
    srcp = jnp.pad(src, ((0, 0), (0, pad))).reshape(_NW, _NCH, _C)
    dstp = jnp.pad(dst, ((0, 0), (0, pad)),
                   constant_values=_TRASH).reshape(_NW, _NCH, _C)

    cp = _sc_degree(dstp)
    y0 = _tc_prep(x)
    sp0 = _sc_seg_sum(y0, srcp, dstp)
    h1, y1 = _tc_layer(sp0, cp, x, W1_0, b1_0, g_0, beta_0, W2_0, b2_0,
                       final=False)
    sp1 = _sc_seg_sum(y1, srcp, dstp)
    h2, y2 = _tc_layer(sp1, cp, h1, W1_1, b1_1, g_1, beta_1, W2_1, b2_1,
                       final=False)
    sp2 = _sc_seg_sum(y2, srcp, dstp)
    (out,) = _tc_layer(sp2, cp, h2, W1_2, b1_2, g_2, beta_2, W2_2, b2_2,
                       final=True)
    return out
